# baseline (speedup 1.0000x reference)
import functools

import jax
import jax.numpy as jnp
from jax.experimental import pallas as pl
from jax.experimental.pallas import tpu as pltpu
from jax.experimental.pallas import tpu_sc as plsc

_NUM_CLUSTERS = 1000
_ROWS = _NUM_CLUSTERS + 1
_EMB = 128
_GW = 128
_NC, _NS = 2, 16
_K = 5


def _sc_gather(tables, idx_ltb, B, T, L):
    mesh = plsc.VectorSubcoreMesh(core_axis_name="c", subcore_axis_name="s")
    nb = B // _GW
    nw = _NC * _NS
    wpw = (T * nb) // nw

    @functools.partial(
        pl.kernel,
        out_type=jax.ShapeDtypeStruct((T, L, B, _EMB), jnp.float32),
        mesh=mesh,
        scratch_types=(
            [pltpu.VMEM_SHARED((_ROWS, _EMB), jnp.float32)] * L
            + [
                pltpu.VMEM((_K, _GW, _EMB), jnp.float32),
                pltpu.VMEM((_K, _GW), jnp.int32),
                pltpu.SemaphoreType.DMA,
                pltpu.SemaphoreType.DMA,
                pltpu.SemaphoreType.DMA,
            ]
        ),
    )
    def k(*refs):
        w_hbms = refs[:L]
        idx_hbm, out_hbm = refs[L], refs[L + 1]
        tables_sh = refs[L + 2 : L + 2 + L]
        stage_buf, idx_buf, sem_i, sem_g, sem_o = refs[L + 2 + L :]

        cid = jax.lax.axis_index("c")
        sid = jax.lax.axis_index("s")
        wid = sid * _NC + cid
        base = wid * wpw

        @pl.when(sid == 0)
        def _():
            for l in range(L):
                pltpu.sync_copy(w_hbms[l], tables_sh[l])

        plsc.subcore_barrier()

        for l in range(L):

            @pl.loop(0, wpw, step=_K)
            def _(w0, l=l):
                loads = []
                for j in range(_K):
                    g = base + w0 + j
                    t, bb = g // nb, g % nb
                    loads.append(
                        pltpu.async_copy(
                            idx_hbm.at[l, t, pl.ds(bb * _GW, _GW)],
                            idx_buf.at[j],
                            sem_i,
                        )
                    )
                for h in loads:
                    h.wait()
                gathers = []
                for j in range(_K):
                    gathers.append(
                        pltpu.async_copy(
                            tables_sh[l].at[idx_buf.at[j]],
                            stage_buf.at[j],
                            sem_g,
                        )
                    )
                for h in gathers:
                    h.wait()
                writes = []
                for j in range(_K):
                    g = base + w0 + j
                    t, bb = g // nb, g % nb
                    writes.append(
                        pltpu.async_copy(
                            stage_buf.at[j],
                            out_hbm.at[t, l, pl.ds(bb * _GW, _GW), :],
                            sem_o,
                        )
                    )
                for h in writes:
                    h.wait()

    return k(*tables, idx_ltb)


def kernel(x, W0, W1, W2):
    B, T, L = x.shape
    idx_ltb = jnp.transpose(x, (2, 1, 0))
    out = _sc_gather((W0, W1, W2), idx_ltb, B, T, L)
    return jnp.transpose(out, (2, 0, 1, 3))


# speedup vs baseline: 1.5046x; 1.5046x over previous
import functools

import jax
import jax.numpy as jnp
from jax.experimental import pallas as pl
from jax.experimental.pallas import tpu as pltpu
from jax.experimental.pallas import tpu_sc as plsc

_NUM_CLUSTERS = 1000
_ROWS = _NUM_CLUSTERS + 1
_EMB = 128
_GW = 128


def _sc_gather(tables, idx_ltb, B, T, L):
    mesh = plsc.VectorSubcoreMesh(core_axis_name="c", subcore_axis_name="s")
    nb = B // _GW

    @functools.partial(
        pl.kernel,
        out_type=jax.ShapeDtypeStruct((T, L, B, _EMB), jnp.float32),
        mesh=mesh,
        scratch_types=[pltpu.VMEM_SHARED((_ROWS, _EMB), jnp.float32)] * L,
    )
    def k(*refs):
        w_hbms = refs[:L]
        idx_hbm, out_hbm = refs[L], refs[L + 1]
        tables_sh = refs[L + 2:]
        sid = jax.lax.axis_index("s")

        @pl.when(sid == 0)
        def _():
            for l in range(L):
                pltpu.sync_copy(w_hbms[l], tables_sh[l])

        plsc.subcore_barrier()

        for l in range(L):
            table_sh = tables_sh[l]

            def body(i_vmem, o_vmem, table_sh=table_sh):
                pltpu.sync_copy(table_sh.at[i_vmem.at[0, 0]], o_vmem.at[0, 0])

            pltpu.emit_pipeline(
                body,
                grid=(T * nb,),
                in_specs=[
                    pl.BlockSpec(
                        (1, 1, _GW),
                        index_map=lambda i, l=l: (l, i // nb, i % nb),
                    )
                ],
                out_specs=[
                    pl.BlockSpec(
                        (1, 1, _GW, _EMB),
                        index_map=lambda i, l=l: (i // nb, l, i % nb, 0),
                    )
                ],
                core_axis_name=("c", "s"),
                dimension_semantics=(pltpu.PARALLEL,),
            )(idx_hbm, out_hbm)

    return k(*tables, idx_ltb)


def kernel(x, W0, W1, W2):
    B, T, L = x.shape
    idx_ltb = jnp.transpose(x, (2, 1, 0))
    out = _sc_gather((W0, W1, W2), idx_ltb, B, T, L)
    return jnp.transpose(out, (2, 0, 1, 3))
